# Initial kernel scaffold; baseline (speedup 1.0000x reference)
import functools

import jax
import jax.numpy as jnp
from jax import lax
from jax.experimental import pallas as pl
from jax.experimental.pallas import tpu as pltpu
from jax.experimental.pallas import tpu_sc as plsc

N = 10000
NPAD = 10240
E = 160000
EPAD = 163840
D = 256
DH = 128
NC, NS = 2, 16
EPS = EPAD // NS
B = 128
ROWS_PER_SUB = NPAD // NS
BM = 1024


def _mm(x, W):
    def body(x_ref, w_ref, o_ref):
        o_ref[0] = jnp.dot(
            x_ref[...],
            w_ref[...],
            preferred_element_type=jnp.float32,
            precision=lax.Precision.HIGHEST,
        )

    return pl.pallas_call(
        body,
        grid=(NPAD // BM, NC),
        in_specs=[
            pl.BlockSpec((BM, D), lambda i, j: (i, 0)),
            pl.BlockSpec((D, DH), lambda i, j: (0, j)),
        ],
        out_specs=pl.BlockSpec((1, BM, DH), lambda i, j: (j, i, 0)),
        out_shape=jax.ShapeDtypeStruct((NC, NPAD, DH), jnp.float32),
    )(x, W)


def _spmm_relu(h_split, src2, dst, w):
    mesh = plsc.VectorSubcoreMesh(core_axis_name="c", subcore_axis_name="s")

    @functools.partial(
        pl.kernel,
        out_type=jax.ShapeDtypeStruct((NPAD, D), jnp.float32),
        mesh=mesh,
        scratch_types=[
            pltpu.VMEM((B,), jnp.int32),
            pltpu.VMEM((B,), jnp.int32),
            pltpu.SMEM((B,), jnp.float32),
            pltpu.VMEM((B, DH), jnp.float32),
            pltpu.VMEM_SHARED((NPAD, DH), jnp.float32),
            pltpu.SemaphoreType.DMA,
        ],
    )
    def k(h_hbm, src_hbm, dst_hbm, w_hbm, out_hbm, srcv, dstv, wv, rows, acc, sem):
        c = lax.axis_index("c")
        s = lax.axis_index("s")

        zero = jnp.zeros((16,), jnp.float32)

        @pl.loop(0, B)
        def _(e):
            for g in range(DH // 16):
                rows[e, pl.ds(g * 16, 16)] = zero

        @pl.loop(0, ROWS_PER_SUB // B)
        def _(i):
            pltpu.sync_copy(rows, acc.at[pl.ds(s * ROWS_PER_SUB + i * B, B), :])

        plsc.subcore_barrier()

        ebase = c * EPAD + s * EPS
        dbase = s * EPS

        @pl.loop(0, EPS // B)
        def _(i):
            pltpu.sync_copy(src_hbm.at[pl.ds(ebase + i * B, B)], srcv)
            pltpu.sync_copy(dst_hbm.at[pl.ds(dbase + i * B, B)], dstv)
            pltpu.sync_copy(w_hbm.at[pl.ds(dbase + i * B, B)], wv)
            pltpu.async_copy(h_hbm.at[srcv], rows, sem).wait()

            @pl.loop(0, B)
            def _(e):
                we = wv[e]
                for g in range(DH // 16):
                    sl = (e, pl.ds(g * 16, 16))
                    rows[sl] = rows[sl] * we

            pltpu.sync_copy(rows, acc.at[dstv], add=True)

        plsc.subcore_barrier()

        @pl.loop(0, ROWS_PER_SUB // B)
        def _(i):
            r0 = s * ROWS_PER_SUB + i * B
            pltpu.sync_copy(acc.at[pl.ds(r0, B), :], rows)

            @pl.loop(0, B)
            def _(e):
                for g in range(DH // 16):
                    sl = (e, pl.ds(g * 16, 16))
                    rows[sl] = jnp.maximum(rows[sl], 0.0)

            pltpu.sync_copy(rows, out_hbm.at[pl.ds(r0, B), pl.ds(c * DH, DH)])

    return k(h_split, src2, dst, w)


def kernel(x, edge_index, edge_weight, W1, W2):
    src = edge_index[0]
    dst = edge_index[1]
    pad = EPAD - E
    srcp = jnp.concatenate([src, jnp.zeros((pad,), jnp.int32)])
    dstp = jnp.concatenate([dst, jnp.zeros((pad,), jnp.int32)])
    wp = jnp.concatenate([edge_weight, jnp.zeros((pad,), jnp.float32)])
    src2 = jnp.concatenate([srcp, srcp + NPAD])
    xp = jnp.pad(x, ((0, NPAD - N), (0, 0)))

    h1 = _mm(xp, W1)
    s1 = _spmm_relu(h1.reshape(NC * NPAD, DH), src2, dstp, wp)
    h2 = _mm(s1, W2)
    s2 = _spmm_relu(h2.reshape(NC * NPAD, DH), src2, dstp, wp)
    return s2[:N]


# speedup vs baseline: 2.2574x; 2.2574x over previous
import functools

import jax
import jax.numpy as jnp
from jax import lax
from jax.experimental import pallas as pl
from jax.experimental.pallas import tpu as pltpu
from jax.experimental.pallas import tpu_sc as plsc

N = 10000
NPAD = 10240
E = 160000
EPAD = 163840
D = 256
DH = 128
NC, NS = 2, 16
EPS = EPAD // NS
B = 128
ROWS_PER_SUB = NPAD // NS
BM = 1024


def _mm(x, W):
    def body(x_ref, w_ref, o_ref):
        o_ref[0] = jnp.dot(
            x_ref[...],
            w_ref[...],
            preferred_element_type=jnp.float32,
            precision=lax.Precision.HIGHEST,
        )

    return pl.pallas_call(
        body,
        grid=(NPAD // BM, NC),
        in_specs=[
            pl.BlockSpec((BM, D), lambda i, j: (i, 0)),
            pl.BlockSpec((D, DH), lambda i, j: (0, j)),
        ],
        out_specs=pl.BlockSpec((1, BM, DH), lambda i, j: (j, i, 0)),
        out_shape=jax.ShapeDtypeStruct((NC, NPAD, DH), jnp.float32),
    )(x, W)


def _spmm_relu(h_split, src2, dst, w):
    mesh = plsc.VectorSubcoreMesh(core_axis_name="c", subcore_axis_name="s")

    @functools.partial(
        pl.kernel,
        out_type=jax.ShapeDtypeStruct((NPAD, D), jnp.float32),
        mesh=mesh,
        scratch_types=[
            pltpu.VMEM((B,), jnp.int32),
            pltpu.VMEM((B,), jnp.int32),
            pltpu.VMEM((B,), jnp.float32),
            pltpu.VMEM((B, DH), jnp.float32),
            pltpu.VMEM_SHARED((NPAD, DH), jnp.float32),
            pltpu.SemaphoreType.DMA,
        ],
    )
    def k(h_hbm, src_hbm, dst_hbm, w_hbm, out_hbm, srcv, dstv, wv, rows, acc, sem):
        c = lax.axis_index("c")
        s = lax.axis_index("s")

        zero = jnp.zeros((16,), jnp.float32)

        @pl.loop(0, B)
        def _(e):
            for g in range(DH // 16):
                rows[e, pl.ds(g * 16, 16)] = zero

        @pl.loop(0, ROWS_PER_SUB // B)
        def _(i):
            pltpu.sync_copy(rows, acc.at[pl.ds(s * ROWS_PER_SUB + i * B, B), :])

        plsc.subcore_barrier()

        ebase = c * EPAD + s * EPS
        dbase = s * EPS

        @pl.loop(0, EPS // B)
        def _(i):
            pltpu.sync_copy(src_hbm.at[pl.ds(ebase + i * B, B)], srcv)
            pltpu.sync_copy(dst_hbm.at[pl.ds(dbase + i * B, B)], dstv)
            pltpu.sync_copy(w_hbm.at[pl.ds(dbase + i * B, B)], wv)
            pltpu.async_copy(h_hbm.at[srcv], rows, sem).wait()

            @pl.loop(0, B // 16)
            def _(g):
                wvec = wv[pl.ds(g * 16, 16)]
                for j in range(16):
                    wsplat = wvec.at[jnp.full((16,), j, jnp.int32)].get(
                        mode="promise_in_bounds"
                    )
                    e = g * 16 + j
                    for k in range(DH // 16):
                        sl = (e, pl.ds(k * 16, 16))
                        rows[sl] = rows[sl] * wsplat

            pltpu.sync_copy(rows, acc.at[dstv], add=True)

        plsc.subcore_barrier()

        @pl.loop(0, ROWS_PER_SUB // B)
        def _(i):
            r0 = s * ROWS_PER_SUB + i * B
            pltpu.sync_copy(acc.at[pl.ds(r0, B), :], rows)

            @pl.loop(0, B)
            def _(e):
                for g in range(DH // 16):
                    sl = (e, pl.ds(g * 16, 16))
                    rows[sl] = jnp.maximum(rows[sl], 0.0)

            pltpu.sync_copy(rows, out_hbm.at[pl.ds(r0, B), pl.ds(c * DH, DH)])

    return k(h_split, src2, dst, w)


def kernel(x, edge_index, edge_weight, W1, W2):
    src = edge_index[0]
    dst = edge_index[1]
    pad = EPAD - E
    srcp = jnp.concatenate([src, jnp.zeros((pad,), jnp.int32)])
    dstp = jnp.concatenate([dst, jnp.zeros((pad,), jnp.int32)])
    wp = jnp.concatenate([edge_weight, jnp.zeros((pad,), jnp.float32)])
    src2 = jnp.concatenate([srcp, srcp + NPAD])
    xp = jnp.pad(x, ((0, NPAD - N), (0, 0)))

    h1 = _mm(xp, W1)
    s1 = _spmm_relu(h1.reshape(NC * NPAD, DH), src2, dstp, wp)
    h2 = _mm(s1, W2)
    s2 = _spmm_relu(h2.reshape(NC * NPAD, DH), src2, dstp, wp)
    return s2[:N]


# speedup vs baseline: 3.4623x; 1.5337x over previous
import functools

import jax
import jax.numpy as jnp
from jax import lax
from jax.experimental import pallas as pl
from jax.experimental.pallas import tpu as pltpu
from jax.experimental.pallas import tpu_sc as plsc

N = 10000
NPAD = 10240
E = 160000
EPAD = 163840
D = 256
DH = 128
NC, NS = 2, 16
EPS = EPAD // NS
B = 128
ROWS_PER_SUB = NPAD // NS
BM = 1024


def _mm(x, W):
    def body(x_ref, w_ref, o_ref):
        o_ref[0] = jnp.dot(
            x_ref[...],
            w_ref[...],
            preferred_element_type=jnp.float32,
            precision=lax.Precision.HIGHEST,
        )

    return pl.pallas_call(
        body,
        grid=(NPAD // BM, NC),
        in_specs=[
            pl.BlockSpec((BM, D), lambda i, j: (i, 0)),
            pl.BlockSpec((D, DH), lambda i, j: (0, j)),
        ],
        out_specs=pl.BlockSpec((1, BM, DH), lambda i, j: (j, i, 0)),
        out_shape=jax.ShapeDtypeStruct((NC, NPAD, DH), jnp.float32),
    )(x, W)


def _spmm_relu(h_split, src2, dst, w):
    mesh = plsc.VectorSubcoreMesh(core_axis_name="c", subcore_axis_name="s")
    NCH = EPS // B

    @functools.partial(
        pl.kernel,
        out_type=jax.ShapeDtypeStruct((NPAD, D), jnp.float32),
        mesh=mesh,
        scratch_types=[
            pltpu.VMEM((4, B), jnp.int32),
            pltpu.VMEM((4, B), jnp.int32),
            pltpu.VMEM((4, B), jnp.float32),
            pltpu.VMEM((2, B, DH), jnp.float32),
            pltpu.VMEM_SHARED((NPAD, DH), jnp.float32),
            pltpu.SemaphoreType.DMA((4,)),
            pltpu.SemaphoreType.DMA((2,)),
            pltpu.SemaphoreType.DMA((2,)),
        ],
    )
    def k(h_hbm, src_hbm, dst_hbm, w_hbm, out_hbm,
          srcv, dstv, wv, rows, acc, si, sg, ss):
        c = lax.axis_index("c")
        s = lax.axis_index("s")

        ebase = c * EPAD + s * EPS
        dbase = s * EPS

        def issue_idx(chunk, r):
            eo = ebase + chunk * B
            do = dbase + chunk * B
            pltpu.async_copy(src_hbm.at[pl.ds(eo, B)], srcv.at[r], si.at[r])
            pltpu.async_copy(dst_hbm.at[pl.ds(do, B)], dstv.at[r], si.at[r])
            pltpu.async_copy(w_hbm.at[pl.ds(do, B)], wv.at[r], si.at[r])

        def wait_idx(r):
            pltpu.make_async_copy(src_hbm.at[pl.ds(0, B)], srcv.at[r], si.at[r]).wait()
            pltpu.make_async_copy(dst_hbm.at[pl.ds(0, B)], dstv.at[r], si.at[r]).wait()
            pltpu.make_async_copy(w_hbm.at[pl.ds(0, B)], wv.at[r], si.at[r]).wait()

        def issue_gather(p, r):
            pltpu.async_copy(h_hbm.at[srcv.at[r]], rows.at[p], sg.at[p])

        def wait_gather(p):
            pltpu.make_async_copy(h_hbm.at[pl.ds(0, B), :], rows.at[p], sg.at[p]).wait()

        def issue_scatter(p, r):
            pltpu.async_copy(rows.at[p], acc.at[dstv.at[r]], ss.at[p], add=True)

        def wait_scatter(p):
            pltpu.make_async_copy(h_hbm.at[pl.ds(0, B), :], rows.at[p], ss.at[p]).wait()

        issue_idx(0, 0)
        issue_idx(1, 1)

        zero = jnp.zeros((16,), jnp.float32)

        @pl.loop(0, B)
        def _(e):
            for g in range(DH // 16):
                rows[0, e, pl.ds(g * 16, 16)] = zero

        @pl.loop(0, ROWS_PER_SUB // B)
        def _(i):
            pltpu.sync_copy(rows.at[0],
                            acc.at[pl.ds(s * ROWS_PER_SUB + i * B, B), :])

        plsc.subcore_barrier()

        wait_idx(0)
        issue_gather(0, 0)

        def scale(p, r):
            @pl.loop(0, B // 16)
            def _(g):
                wvec = wv[r, pl.ds(g * 16, 16)]
                for j in range(16):
                    wsplat = wvec.at[jnp.full((16,), j, jnp.int32)].get(
                        mode="promise_in_bounds"
                    )
                    e = g * 16 + j
                    for kk in range(DH // 16):
                        sl = (p, e, pl.ds(kk * 16, 16))
                        rows[sl] = rows[sl] * wsplat

        def chunk_body(i, u, first):
            p = u % 2
            r = u
            rn = (u + 1) % 4
            r2 = (u + 2) % 4
            wait_gather(p)
            issue_idx(lax.rem(i + 2, NCH), r2)
            if not first:
                wait_scatter(1 - p)
            wait_idx(rn)
            issue_gather(1 - p, rn)
            scale(p, r)
            issue_scatter(p, r)

        for u in range(4):
            chunk_body(u, u, first=(u == 0))

        @pl.loop(1, NCH // 4)
        def _(t):
            for u in range(4):
                chunk_body(t * 4 + u, u, first=False)

        wait_scatter(1)
        wait_gather(0)
        wait_idx(1)

        plsc.subcore_barrier()

        @pl.loop(0, ROWS_PER_SUB // B)
        def _(i):
            r0 = s * ROWS_PER_SUB + i * B
            pltpu.sync_copy(acc.at[pl.ds(r0, B), :], rows.at[0])

            @pl.loop(0, B)
            def _(e):
                for g in range(DH // 16):
                    sl = (0, e, pl.ds(g * 16, 16))
                    rows[sl] = jnp.maximum(rows[sl], 0.0)

            pltpu.sync_copy(rows.at[0],
                            out_hbm.at[pl.ds(r0, B), pl.ds(c * DH, DH)])

    return k(h_split, src2, dst, w)


def kernel(x, edge_index, edge_weight, W1, W2):
    src = edge_index[0]
    dst = edge_index[1]
    pad = EPAD - E
    srcp = jnp.concatenate([src, jnp.zeros((pad,), jnp.int32)])
    dstp = jnp.concatenate([dst, jnp.zeros((pad,), jnp.int32)])
    wp = jnp.concatenate([edge_weight, jnp.zeros((pad,), jnp.float32)])
    src2 = jnp.concatenate([srcp, srcp + NPAD])
    xp = jnp.pad(x, ((0, NPAD - N), (0, 0)))

    h1 = _mm(xp, W1)
    s1 = _spmm_relu(h1.reshape(NC * NPAD, DH), src2, dstp, wp)
    h2 = _mm(s1, W2)
    s2 = _spmm_relu(h2.reshape(NC * NPAD, DH), src2, dstp, wp)
    return s2[:N]
